# baseline (speedup 1.0000x reference)
import dataclasses
import functools

import jax
import jax.numpy as jnp
from jax import lax
from jax.experimental import pallas as pl
from jax.experimental.pallas import tpu as pltpu
from jax.experimental.pallas import tpu_sc as plsc

N = 10000
E = 320000
D = 128

NUM_CORES = 2
NUM_SUBCORES = 16
NW = NUM_CORES * NUM_SUBCORES
CH = 64
CPT = 160
E_PAD = NW * CPT * CH
NCHUNKS = E_PAD // CH
NP = 10240
NROW = NP // 128
MB = 4
NMB = CPT // (2 * MB)
HCH = NCHUNKS // NUM_SUBCORES
HB = 16


@functools.cache
def _vector_mesh():
    return plsc.VectorSubcoreMesh(core_axis_name="c", subcore_axis_name="s")


@functools.cache
def _sc_params():
    cp = pltpu.CompilerParams()
    if "needs_layout_passes" in pltpu.CompilerParams.__dataclass_fields__:
        cp = dataclasses.replace(cp, needs_layout_passes=False)
    return cp


def _rsqrt16(x):
    i = plsc.bitcast(x, jnp.int32)
    i = jnp.int32(0x5F3759DF) - lax.shift_right_arithmetic(i, 1)
    y = plsc.bitcast(i, jnp.float32)
    for _ in range(3):
        y = y * (1.5 - 0.5 * x * y * y)
    return y


def _scale_rows(buf, c0, cw):
    for l in range(16):
        wsc = cw[l]
        for j in range(D // 16):
            sl = (c0 + l, pl.ds(j * 16, 16))
            buf[sl] = buf[sl] * wsc


def _sc_body(feat_hbm, meta_hbm, dst2_hbm, aggp_hbm,
             agg_sh, hist_sh, norm_v, meta_a, meta_b, idbuf,
             rows0, rows1, rows2, rows3,
             gsem0, gsem1, gsem2, gsem3, ssem0, ssem1, ssem2, ssem3, msem):
    cid = lax.axis_index("c")
    sid = lax.axis_index("s")
    wid = cid * NUM_SUBCORES + sid
    rows = (rows0, rows1, rows2, rows3)
    gsems = (gsem0, gsem1, gsem2, gsem3)
    ssems = (ssem0, ssem1, ssem2, ssem3)
    metas = (meta_a, meta_b)

    def drain(buf, sem):
        pltpu.make_async_copy(feat_hbm.at[pl.ds(0, CH)], buf, sem).wait()

    @pl.loop(0, NROW)
    def _(r):
        for j in range(D // 16):
            norm_v[r, pl.ds(j * 16, 16)] = jnp.zeros((16,), jnp.float32)

    @pl.loop(0, CH)
    def _(i):
        for j in range(D // 16):
            rows0[i, pl.ds(j * 16, 16)] = jnp.zeros((16,), jnp.float32)

    for k in range(NP // CH // NUM_SUBCORES):
        z = sid + k * NUM_SUBCORES
        pltpu.async_copy(rows0, agg_sh.at[pl.ds(z * CH, CH)], ssem0)
    for k in range(NP // CH // NUM_SUBCORES):
        drain(rows0, ssem0)

    @pl.when(sid == 0)
    def _():
        pltpu.sync_copy(norm_v, hist_sh)

    @pl.loop(0, NROW, step=16)
    def _(k):
        idbuf[pl.ds(k, 16)] = lax.iota(jnp.int32, 16) + k

    plsc.subcore_barrier()

    ones16 = jnp.full((16,), 1.0, jnp.float32)
    hbase = sid * HCH

    def _hist_block(mref):
        for cc in range(HB):
            @pl.loop(0, CH, step=16)
            def _(c0):
                iv = mref[cc, pl.ds(c0, 16)]
                plsc.addupdate_scatter(
                    norm_v,
                    [lax.shift_right_logical(iv, 7), lax.bitwise_and(iv, 127)],
                    ones16)

    def _mdrain(mref):
        pltpu.make_async_copy(dst2_hbm.at[pl.ds(0, HB)], mref, msem).wait()

    pltpu.async_copy(dst2_hbm.at[pl.ds(hbase, HB)], meta_a, msem)

    @pl.loop(0, HCH // HB // 2)
    def _(u):
        _mdrain(meta_a)
        pltpu.async_copy(dst2_hbm.at[pl.ds(hbase + (2 * u + 1) * HB, HB)],
                         meta_b, msem)
        _hist_block(meta_a)
        _mdrain(meta_b)

        @pl.when(u + 1 < HCH // HB // 2)
        def _():
            pltpu.async_copy(dst2_hbm.at[pl.ds(hbase + (2 * u + 2) * HB, HB)],
                             meta_a, msem)

        _hist_block(meta_b)

    pltpu.sync_copy(norm_v, hist_sh.at[idbuf], add=True)
    plsc.subcore_barrier()

    base = wid * CPT * 4
    pltpu.sync_copy(meta_hbm.at[pl.ds(base, 4 * MB)], meta_a)
    pltpu.async_copy(feat_hbm.at[meta_a.at[0]], rows0, gsem0)
    pltpu.async_copy(feat_hbm.at[meta_a.at[4]], rows1, gsem1)
    pltpu.async_copy(feat_hbm.at[meta_a.at[8]], rows2, gsem2)

    pltpu.sync_copy(hist_sh, norm_v)

    @pl.loop(0, NROW)
    def _(r):
        for j in range(D // 16):
            sl = (r, pl.ds(j * 16, 16))
            norm_v[sl] = _rsqrt16(jnp.maximum(norm_v[sl], 1.0))

    @pl.loop(0, NMB)
    def _(t):
        for j in range(2 * MB):
            cc = j % MB
            b = rows[j % 4]
            mref = metas[j // MB]
            drain(b, gsems[j % 4])

            if j == 0:
                pltpu.async_copy(
                    meta_hbm.at[pl.ds(base + (2 * t + 1) * 4 * MB, 4 * MB)],
                    meta_b, msem)
            if j == 1:
                _mdrain(meta_b)
            if j == 4:
                @pl.when(t + 1 < NMB)
                def _():
                    pltpu.async_copy(
                        meta_hbm.at[pl.ds(base + (2 * t + 2) * 4 * MB,
                                          4 * MB)],
                        meta_a, msem)
            if j == 5:
                @pl.when(t + 1 < NMB)
                def _():
                    _mdrain(meta_a)

            @pl.loop(0, CH, step=16)
            def _(c0):
                iv = mref[4 * cc, pl.ds(c0, 16)]
                nsrc = plsc.load_gather(
                    norm_v,
                    [lax.shift_right_logical(iv, 7), lax.bitwise_and(iv, 127)])
                wv = plsc.bitcast(mref[4 * cc + 2, pl.ds(c0, 16)], jnp.float32)
                _scale_rows(b, c0, wv * nsrc)

            pltpu.async_copy(b, agg_sh.at[mref.at[4 * cc + 1]], ssems[j % 4],
                             add=True)

            j3 = j + 3
            nb = j3 % 4
            m3 = metas[(j3 // MB) % 2]
            idx3 = m3.at[4 * (j3 % MB)]
            if j == 0:
                @pl.when(t > 0)
                def _():
                    drain(rows[nb], ssems[nb])
                pltpu.async_copy(feat_hbm.at[idx3], rows[nb], gsems[nb])
            elif j3 < 2 * MB:
                drain(rows[nb], ssems[nb])
                pltpu.async_copy(feat_hbm.at[idx3], rows[nb], gsems[nb])
            else:
                @pl.when(t + 1 < NMB)
                def _():
                    drain(rows[nb], ssems[nb])
                    pltpu.async_copy(feat_hbm.at[idx3], rows[nb], gsems[nb])

    for k in range(4):
        drain(rows[k], ssems[k])

    plsc.subcore_barrier()

    for k in range(NP // CH // NUM_SUBCORES):
        z = sid + k * NUM_SUBCORES
        buf = rows[k % 2]
        if k >= 2:
            drain(buf, ssems[k % 2])
        pltpu.sync_copy(agg_sh.at[pl.ds(z * CH, CH)], buf)
        zr = lax.shift_right_logical(z, 1)
        zc = lax.bitwise_and(z, 1) * CH

        @pl.loop(0, CH, step=16)
        def _(c0):
            _scale_rows(buf, c0, norm_v[zr, pl.ds(zc + c0, 16)])

        pltpu.async_copy(buf, aggp_hbm.at[pl.ds(cid * NP + z * CH, CH)],
                         ssems[k % 2])
    drain(rows1, ssem1)
    drain(rows0, ssem0)


def _final_body(a0, a1, f, w, b, g, beta, o):
    h = a0[...] + a1[...]
    h = lax.dot_general(h, w[...], (((1,), (1,)), ((), ())),
                        preferred_element_type=jnp.float32) + b[...]
    mu = jnp.mean(h, axis=1, keepdims=True)
    xc = h - mu
    var = jnp.mean(xc * xc, axis=1, keepdims=True)
    h = xc * lax.rsqrt(var + 1e-5) * g[...] + beta[...]
    h = h + f[...]
    o[...] = jnp.maximum(h, 0.0)


def kernel(feat, edge_weight, W, b, ln_gamma, ln_beta, edge_index):
    src = edge_index[0].astype(jnp.int32)
    dst = edge_index[1].astype(jnp.int32)
    w = edge_weight.astype(jnp.float32)

    pad = E_PAD - E
    pad_pos = jnp.arange(pad, dtype=jnp.int32)
    src_p = jnp.concatenate([src, pad_pos % N])
    dst_p = jnp.concatenate([dst, N + pad_pos % (NP - N)])
    w_p = jnp.concatenate([w, jnp.zeros((pad,), jnp.float32)])
    meta = jnp.stack(
        [src_p.reshape(NCHUNKS, CH),
         dst_p.reshape(NCHUNKS, CH),
         lax.bitcast_convert_type(w_p, jnp.int32).reshape(NCHUNKS, CH)],
        axis=1)
    meta = jnp.concatenate(
        [meta, jnp.zeros((NCHUNKS, 1, CH), jnp.int32)],
        axis=1).reshape(NCHUNKS * 4, CH)

    feat_p = jnp.pad(feat, ((0, NP - N), (0, 0)))

    sc_kernel = pl.kernel(
        _sc_body,
        mesh=_vector_mesh(),
        compiler_params=_sc_params(),
        out_type=jax.ShapeDtypeStruct((NUM_CORES * NP, D), jnp.float32),
        scratch_types=[
            pltpu.VMEM_SHARED((NP, D), jnp.float32),
            pltpu.VMEM_SHARED((NROW, D), jnp.float32),
            pltpu.VMEM((NROW, D), jnp.float32),
            pltpu.VMEM((4 * MB, CH), jnp.int32),
            pltpu.VMEM((4 * MB, CH), jnp.int32),
            pltpu.VMEM((NROW,), jnp.int32),
            pltpu.VMEM((CH, D), jnp.float32),
            pltpu.VMEM((CH, D), jnp.float32),
            pltpu.VMEM((CH, D), jnp.float32),
            pltpu.VMEM((CH, D), jnp.float32),
            pltpu.SemaphoreType.DMA,
            pltpu.SemaphoreType.DMA,
            pltpu.SemaphoreType.DMA,
            pltpu.SemaphoreType.DMA,
            pltpu.SemaphoreType.DMA,
            pltpu.SemaphoreType.DMA,
            pltpu.SemaphoreType.DMA,
            pltpu.SemaphoreType.DMA,
            pltpu.SemaphoreType.DMA,
        ],
    )
    aggp = sc_kernel(feat_p, meta, dst_p.reshape(NCHUNKS, CH))

    blk = 1024
    nblk = NP // blk
    row_spec = pl.BlockSpec((blk, D), lambda i: (i, 0))
    vec_spec = pl.BlockSpec((1, D), lambda i: (0, 0))
    out_p = pl.pallas_call(
        _final_body,
        grid=(nblk,),
        in_specs=[row_spec, pl.BlockSpec((blk, D), lambda i: (nblk + i, 0)),
                  row_spec,
                  pl.BlockSpec((D, D), lambda i: (0, 0)),
                  vec_spec, vec_spec, vec_spec],
        out_specs=row_spec,
        out_shape=jax.ShapeDtypeStruct((NP, D), jnp.float32),
    )(aggp, aggp, feat_p, W,
      b.reshape(1, D), ln_gamma.reshape(1, D), ln_beta.reshape(1, D))

    return out_p[:N]


# speedup vs baseline: 1.1253x; 1.1253x over previous
import dataclasses
import functools

import jax
import jax.numpy as jnp
from jax import lax
from jax.experimental import pallas as pl
from jax.experimental.pallas import tpu as pltpu
from jax.experimental.pallas import tpu_sc as plsc

N = 10000
E = 320000
D = 128

NUM_CORES = 2
NUM_SUBCORES = 16
NW = NUM_CORES * NUM_SUBCORES
CHUNK = 128
CPT = 80
E_PAD = NW * CPT * CHUNK
NCHUNKS = E_PAD // CHUNK
HCH = NCHUNKS // NUM_SUBCORES
NP = 10240
NROW = NP // 128
MB = 4
NMB = CPT // MB


@functools.cache
def _vector_mesh():
    return plsc.VectorSubcoreMesh(core_axis_name="c", subcore_axis_name="s")


@functools.cache
def _sc_params():
    cp = pltpu.CompilerParams()
    if "needs_layout_passes" in pltpu.CompilerParams.__dataclass_fields__:
        cp = dataclasses.replace(cp, needs_layout_passes=False)
    return cp


def _rsqrt16(x):
    i = plsc.bitcast(x, jnp.int32)
    i = jnp.int32(0x5F3759DF) - lax.shift_right_arithmetic(i, 1)
    y = plsc.bitcast(i, jnp.float32)
    for _ in range(3):
        y = y * (1.5 - 0.5 * x * y * y)
    return y


def _scale_rows(buf, c0, cw):
    for l in range(16):
        wsc = cw[l]
        for j in range(D // 16):
            sl = (c0 + l, pl.ds(j * 16, 16))
            buf[sl] = buf[sl] * wsc


def _sc_body(feat_hbm, meta_hbm, dst2_hbm, aggp_hbm,
             agg_sh, hist_sh, norm_v, meta_a, meta_b, idbuf,
             rows0, rows1, gsem0, gsem1, ssem0, ssem1, msem):
    cid = lax.axis_index("c")
    sid = lax.axis_index("s")
    wid = cid * NUM_SUBCORES + sid
    rows = (rows0, rows1)
    gsems = (gsem0, gsem1)
    ssems = (ssem0, ssem1)
    metas = (meta_a, meta_b)

    def drain(buf, sem):
        pltpu.make_async_copy(feat_hbm.at[pl.ds(0, CHUNK)], buf, sem).wait()

    @pl.loop(0, NROW)
    def _(r):
        for j in range(D // 16):
            norm_v[r, pl.ds(j * 16, 16)] = jnp.zeros((16,), jnp.float32)

    @pl.loop(0, CHUNK)
    def _(i):
        for j in range(D // 16):
            rows0[i, pl.ds(j * 16, 16)] = jnp.zeros((16,), jnp.float32)

    for k in range(NROW // NUM_SUBCORES):
        z = sid + k * NUM_SUBCORES
        pltpu.async_copy(rows0, agg_sh.at[pl.ds(z * CHUNK, CHUNK)], ssem0)

    @pl.when(sid == 0)
    def _():
        pltpu.sync_copy(norm_v, hist_sh)

    @pl.loop(0, NROW, step=16)
    def _(k):
        idbuf[pl.ds(k, 16)] = lax.iota(jnp.int32, 16) + k

    ones16 = jnp.full((16,), 1.0, jnp.float32)
    HB = MB * 4
    hbase = sid * HCH

    def _hist_block(mref):
        for cc in range(HB):
            @pl.loop(0, CHUNK, step=16)
            def _(c0):
                iv = mref[cc, pl.ds(c0, 16)]
                plsc.addupdate_scatter(
                    norm_v,
                    [lax.shift_right_logical(iv, 7), lax.bitwise_and(iv, 127)],
                    ones16)

    def _mdrain(mref):
        pltpu.make_async_copy(dst2_hbm.at[pl.ds(0, HB)], mref, msem).wait()

    pltpu.async_copy(dst2_hbm.at[pl.ds(hbase, HB)], meta_a, msem)

    @pl.loop(0, HCH // HB // 2)
    def _(u):
        _mdrain(meta_a)
        pltpu.async_copy(dst2_hbm.at[pl.ds(hbase + (2 * u + 1) * HB, HB)],
                         meta_b, msem)
        _hist_block(meta_a)
        _mdrain(meta_b)

        @pl.when(u + 1 < HCH // HB // 2)
        def _():
            pltpu.async_copy(dst2_hbm.at[pl.ds(hbase + (2 * u + 2) * HB, HB)],
                             meta_a, msem)

        _hist_block(meta_b)

    for k in range(NROW // NUM_SUBCORES):
        drain(rows0, ssem0)
    plsc.subcore_barrier()
    pltpu.sync_copy(norm_v, hist_sh.at[idbuf], add=True)
    plsc.subcore_barrier()

    base = wid * CPT * 4
    pltpu.sync_copy(meta_hbm.at[pl.ds(base, MB * 4)], meta_a)
    pltpu.async_copy(feat_hbm.at[meta_a.at[0]], rows0, gsem0)
    pltpu.async_copy(feat_hbm.at[meta_a.at[4]], rows1, gsem1)

    pltpu.sync_copy(hist_sh, norm_v)

    @pl.loop(0, NROW)
    def _(r):
        for j in range(D // 16):
            sl = (r, pl.ds(j * 16, 16))
            norm_v[sl] = _rsqrt16(jnp.maximum(norm_v[sl], 1.0))

    @pl.loop(0, CPT // (2 * MB))
    def _(t):
        for j in range(2 * MB):
            cc = j % MB
            b = rows[j % 2]
            mref = metas[j // MB]
            drain(b, gsems[j % 2])

            if j == 0:
                pltpu.async_copy(
                    meta_hbm.at[pl.ds(base + (2 * t + 1) * MB * 4, MB * 4)],
                    meta_b, msem)
            if j == 2:
                _mdrain(meta_b)
            if j == MB:
                @pl.when(t + 1 < CPT // (2 * MB))
                def _():
                    pltpu.async_copy(
                        meta_hbm.at[pl.ds(base + (2 * t + 2) * MB * 4, MB * 4)],
                        meta_a, msem)
            if j == MB + 2:
                @pl.when(t + 1 < CPT // (2 * MB))
                def _():
                    _mdrain(meta_a)

            @pl.loop(0, CHUNK, step=16)
            def _(c0):
                iv = mref[4 * cc, pl.ds(c0, 16)]
                nsrc = plsc.load_gather(
                    norm_v,
                    [lax.shift_right_logical(iv, 7), lax.bitwise_and(iv, 127)])
                wv = plsc.bitcast(mref[4 * cc + 2, pl.ds(c0, 16)], jnp.float32)
                _scale_rows(b, c0, wv * nsrc)

            pltpu.async_copy(b, agg_sh.at[mref.at[4 * cc + 1]], ssems[j % 2],
                             add=True)
            drain(b, ssems[j % 2])

            j2 = j + 2
            if j2 < 2 * MB:
                pltpu.async_copy(feat_hbm.at[metas[j2 // MB].at[4 * (j2 % MB)]],
                                 b, gsems[j % 2])
            else:
                @pl.when(t + 1 < CPT // (2 * MB))
                def _():
                    pltpu.async_copy(feat_hbm.at[meta_a.at[4 * (j2 % MB)]],
                                     b, gsems[j % 2])

    plsc.subcore_barrier()

    for k in range(NROW // NUM_SUBCORES):
        z = sid + k * NUM_SUBCORES
        buf = rows[k % 2]
        if k >= 2:
            drain(buf, ssems[k % 2])
        pltpu.sync_copy(agg_sh.at[pl.ds(z * CHUNK, CHUNK)], buf)

        @pl.loop(0, CHUNK, step=16)
        def _(c0):
            _scale_rows(buf, c0, norm_v[z, pl.ds(c0, 16)])

        pltpu.async_copy(buf, aggp_hbm.at[pl.ds(cid * NP + z * CHUNK, CHUNK)],
                         ssems[k % 2])
    drain(rows1, ssem1)
    drain(rows0, ssem0)


def _final_body(a0, a1, f, w, b, g, beta, o):
    h = a0[...] + a1[...]
    h = lax.dot_general(h, w[...], (((1,), (1,)), ((), ())),
                        preferred_element_type=jnp.float32) + b[...]
    mu = jnp.mean(h, axis=1, keepdims=True)
    xc = h - mu
    var = jnp.mean(xc * xc, axis=1, keepdims=True)
    h = xc * lax.rsqrt(var + 1e-5) * g[...] + beta[...]
    h = h + f[...]
    o[...] = jnp.maximum(h, 0.0)


def kernel(feat, edge_weight, W, b, ln_gamma, ln_beta, edge_index):
    src = edge_index[0].astype(jnp.int32)
    dst = edge_index[1].astype(jnp.int32)
    w = edge_weight.astype(jnp.float32)

    pad = E_PAD - E
    pad_pos = jnp.arange(pad, dtype=jnp.int32)
    src_p = jnp.concatenate([src, pad_pos % N])
    dst_p = jnp.concatenate([dst, N + pad_pos % (NP - N)])
    w_p = jnp.concatenate([w, jnp.zeros((pad,), jnp.float32)])
    meta = jnp.stack(
        [src_p.reshape(NCHUNKS, CHUNK),
         dst_p.reshape(NCHUNKS, CHUNK),
         lax.bitcast_convert_type(w_p, jnp.int32).reshape(NCHUNKS, CHUNK)],
        axis=1)
    meta = jnp.concatenate(
        [meta, jnp.zeros((NCHUNKS, 1, CHUNK), jnp.int32)],
        axis=1).reshape(NCHUNKS * 4, CHUNK)

    feat_p = jnp.pad(feat, ((0, NP - N), (0, 0)))

    sc_kernel = pl.kernel(
        _sc_body,
        mesh=_vector_mesh(),
        compiler_params=_sc_params(),
        out_type=jax.ShapeDtypeStruct((NUM_CORES * NP, D), jnp.float32),
        scratch_types=[
            pltpu.VMEM_SHARED((NP, D), jnp.float32),
            pltpu.VMEM_SHARED((NROW, D), jnp.float32),
            pltpu.VMEM((NROW, D), jnp.float32),
            pltpu.VMEM((MB * 4, CHUNK), jnp.int32),
            pltpu.VMEM((MB * 4, CHUNK), jnp.int32),
            pltpu.VMEM((NROW,), jnp.int32),
            pltpu.VMEM((CHUNK, D), jnp.float32),
            pltpu.VMEM((CHUNK, D), jnp.float32),
            pltpu.SemaphoreType.DMA,
            pltpu.SemaphoreType.DMA,
            pltpu.SemaphoreType.DMA,
            pltpu.SemaphoreType.DMA,
            pltpu.SemaphoreType.DMA,
        ],
    )
    aggp = sc_kernel(feat_p, meta, dst_p.reshape(NCHUNKS, CHUNK))

    blk = 1024
    nblk = NP // blk
    row_spec = pl.BlockSpec((blk, D), lambda i: (i, 0))
    vec_spec = pl.BlockSpec((1, D), lambda i: (0, 0))
    out_p = pl.pallas_call(
        _final_body,
        grid=(nblk,),
        in_specs=[row_spec, pl.BlockSpec((blk, D), lambda i: (nblk + i, 0)),
                  row_spec,
                  pl.BlockSpec((D, D), lambda i: (0, 0)),
                  vec_spec, vec_spec, vec_spec],
        out_specs=row_spec,
        out_shape=jax.ShapeDtypeStruct((NP, D), jnp.float32),
    )(aggp, aggp, feat_p, W,
      b.reshape(1, D), ln_gamma.reshape(1, D), ln_beta.reshape(1, D))

    return out_p[:N]
